# baseline (speedup 1.0000x reference)
import jax
import jax.numpy as jnp
from jax.experimental import pallas as pl

_ROWS = 6 * 28 * 28 * 16
_LANES = 384
_GRID = 32
_BLOCK_ROWS = _ROWS // _GRID


def _copy_body(src_ref, dst_ref):
    dst_ref[...] = src_ref[...]


def kernel(data):
    t, b, c, h, w = data.shape
    x = jnp.transpose(data, (0, 3, 4, 1, 2)).reshape(_ROWS, _LANES)
    out = pl.pallas_call(
        _copy_body,
        grid=(_GRID,),
        in_specs=[pl.BlockSpec((_BLOCK_ROWS, _LANES), lambda i: (i, 0))],
        out_specs=pl.BlockSpec((_BLOCK_ROWS, _LANES), lambda i: (i, 0)),
        out_shape=jax.ShapeDtypeStruct((_ROWS, _LANES), data.dtype),
    )(x)
    return jnp.transpose(out.reshape(t, h, w, b, c), (0, 3, 4, 1, 2))


# speedup vs baseline: 1.0316x; 1.0316x over previous
import jax
import jax.numpy as jnp
from jax.experimental import pallas as pl

_ROWS = 6 * 28 * 28 * 16
_LANES = 384
_GRID = 8
_BLOCK_ROWS = _ROWS // _GRID


def _copy_body(src_ref, dst_ref):
    dst_ref[...] = src_ref[...]


def kernel(data):
    t, b, c, h, w = data.shape
    x = jnp.transpose(data, (0, 3, 4, 1, 2)).reshape(_ROWS, _LANES)
    out = pl.pallas_call(
        _copy_body,
        grid=(_GRID,),
        in_specs=[pl.BlockSpec((_BLOCK_ROWS, _LANES), lambda i: (i, 0))],
        out_specs=pl.BlockSpec((_BLOCK_ROWS, _LANES), lambda i: (i, 0)),
        out_shape=jax.ShapeDtypeStruct((_ROWS, _LANES), data.dtype),
    )(x)
    return jnp.transpose(out.reshape(t, h, w, b, c), (0, 3, 4, 1, 2))


# speedup vs baseline: 1.0319x; 1.0003x over previous
import jax
import jax.numpy as jnp
from jax.experimental import pallas as pl

_ROWS = 6 * 28 * 28 * 16
_LANES = 384
_GRID = 12
_BLOCK_ROWS = _ROWS // _GRID


def _copy_body(src_ref, dst_ref):
    dst_ref[...] = src_ref[...]


def kernel(data):
    t, b, c, h, w = data.shape
    x = jnp.transpose(data, (0, 3, 4, 1, 2)).reshape(_ROWS, _LANES)
    out = pl.pallas_call(
        _copy_body,
        grid=(_GRID,),
        in_specs=[pl.BlockSpec((_BLOCK_ROWS, _LANES), lambda i: (i, 0))],
        out_specs=pl.BlockSpec((_BLOCK_ROWS, _LANES), lambda i: (i, 0)),
        out_shape=jax.ShapeDtypeStruct((_ROWS, _LANES), data.dtype),
    )(x)
    return jnp.transpose(out.reshape(t, h, w, b, c), (0, 3, 4, 1, 2))
